# baseline (speedup 1.0000x reference)
import functools

import numpy as _np

import jax
import jax.numpy as jnp
from jax import lax
from jax.experimental import pallas as pl
from jax.experimental.pallas import tpu as pltpu
from jax.experimental.pallas import tpu_sc as plsc

E = 8
D = 1024
FF = 2048
T = 2048
A = 2 * T
B = 512
G = A // B + E
P = G * B
F = 512
NFF = FF // F

NW = 32
TPW = T // NW


def _router_body(x_ref, rw_ref, ut_ref, pos_ref, w2_ref, te_ref, xg_ref,
                 valid_ref, pj_ref):
    lg = lax.dot_general(rw_ref[...], x_ref[...], (((0,), (1,)), ((), ())),
                         preferred_element_type=jnp.float32)
    e_iota = lax.broadcasted_iota(jnp.int32, (E, T), 0)
    m1 = jnp.max(lg, axis=0, keepdims=True)
    i1 = jnp.min(jnp.where(lg == m1, e_iota, E), axis=0, keepdims=True)
    masked = jnp.where(e_iota == i1, -jnp.inf, lg)
    m2 = jnp.max(masked, axis=0, keepdims=True)
    i2 = jnp.min(jnp.where(masked == m2, e_iota, E), axis=0, keepdims=True)
    w1 = 1.0 / (1.0 + jnp.exp(m2 - m1))
    w2 = 1.0 - w1

    sel1 = (e_iota == i1).astype(jnp.float32)
    sel2 = (e_iota == i2).astype(jnp.float32)
    s = sel1 + sel2

    csum = lax.dot_general(s.astype(jnp.bfloat16), ut_ref[...],
                           (((1,), (0,)), ((), ())),
                           preferred_element_type=jnp.float32)
    counts = csum[:, T - 1:T] + s[:, T - 1:T]

    cnt_pad = jnp.ceil(counts / B) * B
    e_row = lax.broadcasted_iota(jnp.int32, (E, E), 0)
    e_col = lax.broadcasted_iota(jnp.int32, (E, E), 1)
    lt_e = (e_col < e_row).astype(jnp.float32)
    base = lax.dot_general(lt_e, cnt_pad, (((1,), (0,)), ((), ())),
                           preferred_element_type=jnp.float32)
    total_pad = base[E - 1, 0] + cnt_pad[E - 1, 0]

    rank1 = jnp.sum(csum * sel1, axis=0, keepdims=True)
    rank2 = jnp.sum(csum * sel2, axis=0, keepdims=True)
    base1 = jnp.sum(base * sel1, axis=0, keepdims=True)
    base2 = jnp.sum(base * sel2, axis=0, keepdims=True)
    pos_ref[0:1, :] = (base1 + rank1).astype(jnp.int32)
    pos_ref[1:2, :] = (base2 + rank2).astype(jnp.int32)
    w2_ref[0] = jnp.broadcast_to(jnp.reshape(w1, (T, 1)), (T, 128))
    w2_ref[1] = jnp.broadcast_to(jnp.reshape(w2, (T, 1)), (T, 128))

    nvalid = total_pad / B
    g_iota = lax.broadcasted_iota(jnp.int32, (1, G), 1).astype(jnp.float32)
    validg = g_iota < nvalid
    gs = jnp.minimum(g_iota * B, total_pad - B)
    ge = jnp.sum((jnp.broadcast_to(base, (E, G)) <= gs).astype(jnp.int32),
                 axis=0, keepdims=True) - 1
    xg = jnp.minimum(g_iota, nvalid - 1.0)
    te_ref[...] = ge
    xg_ref[...] = xg.astype(jnp.int32)
    valid_ref[...] = validg.astype(jnp.int32)

    e_iota_g = lax.broadcasted_iota(jnp.int32, (E, G), 0)
    baseg = jnp.sum(jnp.broadcast_to(base, (E, G)) * (e_iota_g == ge),
                    axis=0, keepdims=True)
    run = g_iota - baseg / B
    par = run - 2.0 * jnp.floor(run / 2.0)
    parlast = jnp.sum(par * (g_iota == nvalid - 1.0), axis=1, keepdims=True)
    pj = jnp.where(validg, par, 2.0 + parlast)
    pj_ref[...] = pj.astype(jnp.int32)


def _router(flat, router_w, ut_const):
    return pl.pallas_call(
        _router_body,
        in_specs=[
            pl.BlockSpec((T, D), lambda: (0, 0)),
            pl.BlockSpec((D, E), lambda: (0, 0)),
            pl.BlockSpec((T, T), lambda: (0, 0)),
        ],
        out_specs=[
            pl.BlockSpec((2, T), lambda: (0, 0)),
            pl.BlockSpec((2, T, 128), lambda: (0, 0, 0)),
            pl.BlockSpec((1, G), lambda: (0, 0)),
            pl.BlockSpec((1, G), lambda: (0, 0)),
            pl.BlockSpec((1, G), lambda: (0, 0)),
            pl.BlockSpec((1, G), lambda: (0, 0)),
        ],
        out_shape=[
            jax.ShapeDtypeStruct((2, T), jnp.int32),
            jax.ShapeDtypeStruct((2, T, 128), jnp.float32),
            jax.ShapeDtypeStruct((1, G), jnp.int32),
            jax.ShapeDtypeStruct((1, G), jnp.int32),
            jax.ShapeDtypeStruct((1, G), jnp.int32),
            jax.ShapeDtypeStruct((1, G), jnp.int32),
        ],
    )(flat, router_w, ut_const)


def _dispatch_sc(flat, pos, w2):
    mesh = plsc.VectorSubcoreMesh(core_axis_name="c", subcore_axis_name="s")

    @functools.partial(
        pl.kernel, mesh=mesh,
        out_type=(
            jax.ShapeDtypeStruct((P, D), jnp.float32),
            jax.ShapeDtypeStruct((P, 128), jnp.float32),
        ),
        scratch_types=[
            pltpu.VMEM((TPW,), jnp.int32),
            pltpu.VMEM((TPW,), jnp.int32),
            pltpu.VMEM((TPW, 128), jnp.float32),
            pltpu.VMEM((TPW, 128), jnp.float32),
            pltpu.VMEM((TPW, D), jnp.float32),
            pltpu.SemaphoreType.DMA,
        ],
    )
    def k(flat_hbm, pos_hbm, w2_hbm, xs_hbm, ws_hbm,
          idx0_v, idx1_v, wr0_v, wr1_v, rows_v, sem):
        wid = lax.axis_index("s") * 2 + lax.axis_index("c")
        tbase = wid * TPW
        l0 = pltpu.async_copy(pos_hbm.at[0, pl.ds(tbase, TPW)], idx0_v, sem)
        l1 = pltpu.async_copy(pos_hbm.at[1, pl.ds(tbase, TPW)], idx1_v, sem)
        l2 = pltpu.async_copy(w2_hbm.at[0, pl.ds(tbase, TPW)], wr0_v, sem)
        l3 = pltpu.async_copy(w2_hbm.at[1, pl.ds(tbase, TPW)], wr1_v, sem)
        l4 = pltpu.async_copy(flat_hbm.at[pl.ds(tbase, TPW)], rows_v, sem)
        l0.wait()
        l1.wait()
        l2.wait()
        l3.wait()
        l4.wait()
        c0 = pltpu.async_copy(rows_v, xs_hbm.at[idx0_v], sem)
        c1 = pltpu.async_copy(rows_v, xs_hbm.at[idx1_v], sem)
        c2 = pltpu.async_copy(wr0_v, ws_hbm.at[idx0_v], sem)
        c3 = pltpu.async_copy(wr1_v, ws_hbm.at[idx1_v], sem)
        c0.wait()
        c1.wait()
        c2.wait()
        c3.wait()

    return k(flat, pos, w2)


def _gemm_body(te_ref, xg_ref, valid_ref, pj_ref, xs_ref, wg_ref, wu_ref,
               wd_ref, ws_ref, y_ref, acc_ref):
    g = pl.program_id(0)
    j = pl.program_id(1)

    @pl.when(valid_ref[g] == 1)
    def _():
        x = xs_ref[...]
        gate = lax.dot_general(x, wg_ref[0].astype(jnp.bfloat16),
                               (((1,), (0,)), ((), ())),
                               preferred_element_type=jnp.float32)
        up = lax.dot_general(x, wu_ref[0].astype(jnp.bfloat16),
                             (((1,), (0,)), ((), ())),
                             preferred_element_type=jnp.float32)
        gated = (gate * lax.logistic(gate)) * up
        contrib = lax.dot_general(gated.astype(jnp.bfloat16),
                                  wd_ref[0].astype(jnp.bfloat16),
                                  (((1,), (0,)), ((), ())),
                                  preferred_element_type=jnp.float32)

        @pl.when(j == 0)
        def _():
            acc_ref[...] = jnp.zeros_like(acc_ref)

        acc_ref[...] += contrib

        @pl.when(j == NFF - 1)
        def _():
            y_ref[...] = acc_ref[...] * ws_ref[:, 0:1]


def _gemm(te, xg, valid, pj, xs, gate_up_proj, down_proj, ws):
    def jc(j, pjg):
        return jnp.where(pjg == 0, j,
                         jnp.where(pjg == 1, NFF - 1 - j,
                                   (3 - pjg) * (NFF - 1)))

    grid_spec = pltpu.PrefetchScalarGridSpec(
        num_scalar_prefetch=4,
        grid=(G, NFF),
        in_specs=[
            pl.BlockSpec((B, D), lambda g, j, te, xg, v, pj: (xg[g], 0)),
            pl.BlockSpec((1, D, F),
                         lambda g, j, te, xg, v, pj: (te[g], 0, jc(j, pj[g]))),
            pl.BlockSpec((1, D, F),
                         lambda g, j, te, xg, v, pj:
                         (te[g], 0, jc(j, pj[g]) + NFF)),
            pl.BlockSpec((1, F, D),
                         lambda g, j, te, xg, v, pj: (te[g], jc(j, pj[g]), 0)),
            pl.BlockSpec((B, 128), lambda g, j, te, xg, v, pj: (xg[g], 0)),
        ],
        out_specs=pl.BlockSpec((B, D), lambda g, j, te, xg, v, pj: (xg[g], 0)),
        scratch_shapes=[pltpu.VMEM((B, D), jnp.float32)],
    )
    return pl.pallas_call(
        _gemm_body,
        grid_spec=grid_spec,
        out_shape=jax.ShapeDtypeStruct((P, D), jnp.float32),
        compiler_params=pltpu.CompilerParams(
            dimension_semantics=("arbitrary", "arbitrary"),
        ),
    )(te, xg, valid, pj, xs, gate_up_proj, gate_up_proj, down_proj, ws)


_CH = 32


def _combine_sc(y, pos):
    mesh = plsc.VectorSubcoreMesh(core_axis_name="c", subcore_axis_name="s")

    @functools.partial(
        pl.kernel, mesh=mesh,
        out_type=jax.ShapeDtypeStruct((T, D), jnp.float32),
        scratch_types=[
            pltpu.VMEM((_CH,), jnp.int32),
            pltpu.VMEM((_CH,), jnp.int32),
            pltpu.VMEM((_CH, D), jnp.float32),
            pltpu.VMEM((_CH, D), jnp.float32),
            pltpu.SemaphoreType.DMA,
        ],
    )
    def k(y_hbm, pos_hbm, out_hbm, idx0_v, idx1_v, y0_v, y1_v, sem):
        wid = lax.axis_index("s") * 2 + lax.axis_index("c")
        for ch in range(TPW // _CH):
            tbase = wid * TPW + ch * _CH
            pltpu.sync_copy(pos_hbm.at[0, pl.ds(tbase, _CH)], idx0_v)
            pltpu.sync_copy(pos_hbm.at[1, pl.ds(tbase, _CH)], idx1_v)
            c0 = pltpu.async_copy(y_hbm.at[idx0_v], y0_v, sem)
            c1 = pltpu.async_copy(y_hbm.at[idx1_v], y1_v, sem)
            c0.wait()
            c1.wait()

            def row(r, _):
                for c in range(D // 16):
                    y0_v[r, pl.ds(c * 16, 16)] += y1_v[r, pl.ds(c * 16, 16)]
                return 0

            lax.fori_loop(0, _CH, row, 0)
            pltpu.sync_copy(y0_v, out_hbm.at[pl.ds(tbase, _CH)])

    return k(y, pos)


@jax.jit
def kernel(hidden_states, router_w, gate_up_proj, down_proj):
    b, s, d = hidden_states.shape
    flat = hidden_states.reshape(-1, d)
    ut_const = jnp.asarray(_np.triu(_np.ones((T, T), _np.float32), k=1),
                           dtype=jnp.bfloat16)
    pos, w2, te, xg, valid, pj = _router(flat, router_w, ut_const)
    xs, ws = _dispatch_sc(flat, pos, w2)
    y = _gemm(te.reshape(G), xg.reshape(G), valid.reshape(G), pj.reshape(G),
              xs, gate_up_proj, down_proj, ws)
    out = _combine_sc(y, pos)
    return out.reshape(b, s, d)


# speedup vs baseline: 1.1012x; 1.1012x over previous
import functools

import numpy as _np

import jax
import jax.numpy as jnp
from jax import lax
from jax.experimental import pallas as pl
from jax.experimental.pallas import tpu as pltpu
from jax.experimental.pallas import tpu_sc as plsc

E = 8
D = 1024
FF = 2048
T = 2048
A = 2 * T
B = 512
G = A // B + E
P = G * B
F = 1024
NFF = FF // F

NW = 32
TPW = T // NW


def _router_body(x_ref, rw_ref, ut_ref, pos_ref, w2_ref, te_ref, xg_ref,
                 valid_ref, pj_ref):
    lg = lax.dot_general(rw_ref[...], x_ref[...], (((0,), (1,)), ((), ())),
                         preferred_element_type=jnp.float32)
    e_iota = lax.broadcasted_iota(jnp.int32, (E, T), 0)
    m1 = jnp.max(lg, axis=0, keepdims=True)
    i1 = jnp.min(jnp.where(lg == m1, e_iota, E), axis=0, keepdims=True)
    masked = jnp.where(e_iota == i1, -jnp.inf, lg)
    m2 = jnp.max(masked, axis=0, keepdims=True)
    i2 = jnp.min(jnp.where(masked == m2, e_iota, E), axis=0, keepdims=True)
    w1 = 1.0 / (1.0 + jnp.exp(m2 - m1))
    w2 = 1.0 - w1

    sel1 = (e_iota == i1).astype(jnp.float32)
    sel2 = (e_iota == i2).astype(jnp.float32)
    s = sel1 + sel2

    csum = lax.dot_general(s.astype(jnp.bfloat16), ut_ref[...],
                           (((1,), (0,)), ((), ())),
                           preferred_element_type=jnp.float32)
    counts = csum[:, T - 1:T] + s[:, T - 1:T]

    cnt_pad = jnp.ceil(counts / B) * B
    e_row = lax.broadcasted_iota(jnp.int32, (E, E), 0)
    e_col = lax.broadcasted_iota(jnp.int32, (E, E), 1)
    lt_e = (e_col < e_row).astype(jnp.float32)
    base = lax.dot_general(lt_e, cnt_pad, (((1,), (0,)), ((), ())),
                           preferred_element_type=jnp.float32)
    total_pad = base[E - 1, 0] + cnt_pad[E - 1, 0]

    rank1 = jnp.sum(csum * sel1, axis=0, keepdims=True)
    rank2 = jnp.sum(csum * sel2, axis=0, keepdims=True)
    base1 = jnp.sum(base * sel1, axis=0, keepdims=True)
    base2 = jnp.sum(base * sel2, axis=0, keepdims=True)
    pos_ref[0:1, :] = (base1 + rank1).astype(jnp.int32)
    pos_ref[1:2, :] = (base2 + rank2).astype(jnp.int32)
    w2_ref[0] = jnp.broadcast_to(jnp.reshape(w1, (T, 1)), (T, 128))
    w2_ref[1] = jnp.broadcast_to(jnp.reshape(w2, (T, 1)), (T, 128))

    nvalid = total_pad / B
    g_iota = lax.broadcasted_iota(jnp.int32, (1, G), 1).astype(jnp.float32)
    validg = g_iota < nvalid
    gs = jnp.minimum(g_iota * B, total_pad - B)
    ge = jnp.sum((jnp.broadcast_to(base, (E, G)) <= gs).astype(jnp.int32),
                 axis=0, keepdims=True) - 1
    xg = jnp.minimum(g_iota, nvalid - 1.0)
    te_ref[...] = ge
    xg_ref[...] = xg.astype(jnp.int32)
    valid_ref[...] = validg.astype(jnp.int32)

    e_iota_g = lax.broadcasted_iota(jnp.int32, (E, G), 0)
    baseg = jnp.sum(jnp.broadcast_to(base, (E, G)) * (e_iota_g == ge),
                    axis=0, keepdims=True)
    run = g_iota - baseg / B
    par = run - 2.0 * jnp.floor(run / 2.0)
    parlast = jnp.sum(par * (g_iota == nvalid - 1.0), axis=1, keepdims=True)
    pj = jnp.where(validg, par, 2.0 + parlast)
    pj_ref[...] = pj.astype(jnp.int32)


def _router(flat, router_w, ut_const):
    return pl.pallas_call(
        _router_body,
        in_specs=[
            pl.BlockSpec((T, D), lambda: (0, 0)),
            pl.BlockSpec((D, E), lambda: (0, 0)),
            pl.BlockSpec((T, T), lambda: (0, 0)),
        ],
        out_specs=[
            pl.BlockSpec((2, T), lambda: (0, 0)),
            pl.BlockSpec((2, T, 128), lambda: (0, 0, 0)),
            pl.BlockSpec((1, G), lambda: (0, 0)),
            pl.BlockSpec((1, G), lambda: (0, 0)),
            pl.BlockSpec((1, G), lambda: (0, 0)),
            pl.BlockSpec((1, G), lambda: (0, 0)),
        ],
        out_shape=[
            jax.ShapeDtypeStruct((2, T), jnp.int32),
            jax.ShapeDtypeStruct((2, T, 128), jnp.float32),
            jax.ShapeDtypeStruct((1, G), jnp.int32),
            jax.ShapeDtypeStruct((1, G), jnp.int32),
            jax.ShapeDtypeStruct((1, G), jnp.int32),
            jax.ShapeDtypeStruct((1, G), jnp.int32),
        ],
    )(flat, router_w, ut_const)


def _dispatch_sc(flat, pos, w2):
    mesh = plsc.VectorSubcoreMesh(core_axis_name="c", subcore_axis_name="s")

    @functools.partial(
        pl.kernel, mesh=mesh,
        out_type=(
            jax.ShapeDtypeStruct((P, D), jnp.float32),
            jax.ShapeDtypeStruct((P, 128), jnp.float32),
        ),
        scratch_types=[
            pltpu.VMEM((TPW,), jnp.int32),
            pltpu.VMEM((TPW,), jnp.int32),
            pltpu.VMEM((TPW, 128), jnp.float32),
            pltpu.VMEM((TPW, 128), jnp.float32),
            pltpu.VMEM((TPW, D), jnp.float32),
            pltpu.SemaphoreType.DMA,
        ],
    )
    def k(flat_hbm, pos_hbm, w2_hbm, xs_hbm, ws_hbm,
          idx0_v, idx1_v, wr0_v, wr1_v, rows_v, sem):
        wid = lax.axis_index("s") * 2 + lax.axis_index("c")
        tbase = wid * TPW
        l0 = pltpu.async_copy(pos_hbm.at[0, pl.ds(tbase, TPW)], idx0_v, sem)
        l1 = pltpu.async_copy(pos_hbm.at[1, pl.ds(tbase, TPW)], idx1_v, sem)
        l2 = pltpu.async_copy(w2_hbm.at[0, pl.ds(tbase, TPW)], wr0_v, sem)
        l3 = pltpu.async_copy(w2_hbm.at[1, pl.ds(tbase, TPW)], wr1_v, sem)
        l4 = pltpu.async_copy(flat_hbm.at[pl.ds(tbase, TPW)], rows_v, sem)
        l0.wait()
        l1.wait()
        l2.wait()
        l3.wait()
        l4.wait()
        c0 = pltpu.async_copy(rows_v, xs_hbm.at[idx0_v], sem)
        c1 = pltpu.async_copy(rows_v, xs_hbm.at[idx1_v], sem)
        c2 = pltpu.async_copy(wr0_v, ws_hbm.at[idx0_v], sem)
        c3 = pltpu.async_copy(wr1_v, ws_hbm.at[idx1_v], sem)
        c0.wait()
        c1.wait()
        c2.wait()
        c3.wait()

    return k(flat, pos, w2)


def _gemm_body(te_ref, xg_ref, valid_ref, pj_ref, xs_ref, wg_ref, wu_ref,
               wd_ref, ws_ref, y_ref, acc_ref):
    g = pl.program_id(0)
    j = pl.program_id(1)

    @pl.when(valid_ref[g] == 1)
    def _():
        x = xs_ref[...]
        gate = lax.dot_general(x, wg_ref[0].astype(jnp.bfloat16),
                               (((1,), (0,)), ((), ())),
                               preferred_element_type=jnp.float32)
        up = lax.dot_general(x, wu_ref[0].astype(jnp.bfloat16),
                             (((1,), (0,)), ((), ())),
                             preferred_element_type=jnp.float32)
        gated = (gate * lax.logistic(gate)) * up
        contrib = lax.dot_general(gated.astype(jnp.bfloat16),
                                  wd_ref[0].astype(jnp.bfloat16),
                                  (((1,), (0,)), ((), ())),
                                  preferred_element_type=jnp.float32)

        @pl.when(j == 0)
        def _():
            acc_ref[...] = jnp.zeros_like(acc_ref)

        acc_ref[...] += contrib

        @pl.when(j == NFF - 1)
        def _():
            y_ref[...] = acc_ref[...] * ws_ref[:, 0:1]


def _gemm(te, xg, valid, pj, xs, gate_up_proj, down_proj, ws):
    def jc(j, pjg):
        return jnp.where(pjg == 0, j,
                         jnp.where(pjg == 1, NFF - 1 - j,
                                   (3 - pjg) * (NFF - 1)))

    grid_spec = pltpu.PrefetchScalarGridSpec(
        num_scalar_prefetch=4,
        grid=(G, NFF),
        in_specs=[
            pl.BlockSpec((B, D), lambda g, j, te, xg, v, pj: (xg[g], 0)),
            pl.BlockSpec((1, D, F),
                         lambda g, j, te, xg, v, pj: (te[g], 0, jc(j, pj[g]))),
            pl.BlockSpec((1, D, F),
                         lambda g, j, te, xg, v, pj:
                         (te[g], 0, jc(j, pj[g]) + NFF)),
            pl.BlockSpec((1, F, D),
                         lambda g, j, te, xg, v, pj: (te[g], jc(j, pj[g]), 0)),
            pl.BlockSpec((B, 128), lambda g, j, te, xg, v, pj: (xg[g], 0)),
        ],
        out_specs=pl.BlockSpec((B, D), lambda g, j, te, xg, v, pj: (xg[g], 0)),
        scratch_shapes=[pltpu.VMEM((B, D), jnp.float32)],
    )
    return pl.pallas_call(
        _gemm_body,
        grid_spec=grid_spec,
        out_shape=jax.ShapeDtypeStruct((P, D), jnp.float32),
        compiler_params=pltpu.CompilerParams(
            dimension_semantics=("arbitrary", "arbitrary"),
        ),
    )(te, xg, valid, pj, xs, gate_up_proj, gate_up_proj, down_proj, ws)


_CH = 32


def _combine_sc(y, pos):
    mesh = plsc.VectorSubcoreMesh(core_axis_name="c", subcore_axis_name="s")

    @functools.partial(
        pl.kernel, mesh=mesh,
        out_type=jax.ShapeDtypeStruct((T, D), jnp.float32),
        scratch_types=[
            pltpu.VMEM((_CH,), jnp.int32),
            pltpu.VMEM((_CH,), jnp.int32),
            pltpu.VMEM((_CH, D), jnp.float32),
            pltpu.VMEM((_CH, D), jnp.float32),
            pltpu.SemaphoreType.DMA,
        ],
    )
    def k(y_hbm, pos_hbm, out_hbm, idx0_v, idx1_v, y0_v, y1_v, sem):
        wid = lax.axis_index("s") * 2 + lax.axis_index("c")
        for ch in range(TPW // _CH):
            tbase = wid * TPW + ch * _CH
            pltpu.sync_copy(pos_hbm.at[0, pl.ds(tbase, _CH)], idx0_v)
            pltpu.sync_copy(pos_hbm.at[1, pl.ds(tbase, _CH)], idx1_v)
            c0 = pltpu.async_copy(y_hbm.at[idx0_v], y0_v, sem)
            c1 = pltpu.async_copy(y_hbm.at[idx1_v], y1_v, sem)
            c0.wait()
            c1.wait()

            def row(r, _):
                for c in range(D // 16):
                    y0_v[r, pl.ds(c * 16, 16)] += y1_v[r, pl.ds(c * 16, 16)]
                return 0

            lax.fori_loop(0, _CH, row, 0)
            pltpu.sync_copy(y0_v, out_hbm.at[pl.ds(tbase, _CH)])

    return k(y, pos)


@jax.jit
def kernel(hidden_states, router_w, gate_up_proj, down_proj):
    b, s, d = hidden_states.shape
    flat = hidden_states.reshape(-1, d)
    ut_const = jnp.asarray(_np.triu(_np.ones((T, T), _np.float32), k=1),
                           dtype=jnp.bfloat16)
    pos, w2, te, xg, valid, pj = _router(flat, router_w, ut_const)
    xs, ws = _dispatch_sc(flat, pos, w2)
    y = _gemm(te.reshape(G), xg.reshape(G), valid.reshape(G), pj.reshape(G),
              xs, gate_up_proj, down_proj, ws)
    out = _combine_sc(y, pos)
    return out.reshape(b, s, d)
